# Initial kernel scaffold; baseline (speedup 1.0000x reference)
import jax
import jax.numpy as jnp
from jax.experimental import pallas as pl


def kernel(input_ids, word_emb, pos_emb, ln_w, ln_b):
    raise NotImplementedError("write your pallas kernel here")


# speedup vs baseline: 1.5789x; 1.5789x over previous
import functools

import jax
import jax.numpy as jnp
from jax import lax
from jax.experimental import pallas as pl
from jax.experimental.pallas import tpu as pltpu
from jax.experimental.pallas import tpu_sc as plsc

_VOCAB = 26
_HID = 640
_PAD = 1
_MASK = 24
_SPAD = 1024
_EPS = 1e-12
_MASK_RATIO_TRAIN = 0.15 * 0.8

_NC = 2
_NS = 16
_NW = _NC * _NS
_CH = 128

_TBLK = 512


def _stage1_body(ids_ref, pos_ref, a_ref, e_ref):
    ids = ids_ref[...]
    maskf = (ids != _PAD).astype(jnp.float32)
    s = ids.shape[1]
    row = lax.broadcasted_iota(jnp.int32, (s, s), 0)
    col = lax.broadcasted_iota(jnp.int32, (s, s), 1)
    tri = (row <= col).astype(jnp.bfloat16)
    inc = jnp.dot(maskf.astype(jnp.bfloat16), tri,
                  preferred_element_type=jnp.float32)
    pos_ref[...] = (inc * maskf + 2.0).astype(jnp.int32)
    ismask = ids == _MASK
    nmask = jnp.sum(ismask.astype(jnp.float32), axis=1, keepdims=True)
    srclen = jnp.sum(maskf, axis=1, keepdims=True)
    scale = (1.0 - _MASK_RATIO_TRAIN) / (1.0 - nmask / srclen)
    a_ref[...] = jnp.where(ismask, 0.0, scale)
    e_ref[...] = maskf


def _main_body(ids_ref, a_ref, e_ref, w_ref, lnw_ref, lnb_ref, pg_ref, out_ref):
    ids = ids_ref[...]
    voc = lax.broadcasted_iota(jnp.int32, (1, _VOCAB), 1)
    oh = (ids == voc).astype(jnp.float32)
    wg = jnp.dot(oh, w_ref[...], preferred_element_type=jnp.float32)
    x = a_ref[...] * wg + pg_ref[...]
    mu = jnp.mean(x, axis=1, keepdims=True)
    xc = x - mu
    var = jnp.mean(xc * xc, axis=1, keepdims=True)
    y = xc * lax.rsqrt(var + _EPS) * lnw_ref[...] + lnb_ref[...]
    out_ref[...] = y * e_ref[...]


def _sc_gather(table, idx):
    n = idx.shape[0]
    b_per_w = n // _NW
    mesh = plsc.VectorSubcoreMesh(core_axis_name="c", subcore_axis_name="s")

    @functools.partial(
        pl.kernel,
        mesh=mesh,
        out_type=jax.ShapeDtypeStruct((n, _HID), jnp.float32),
        scratch_types=[
            pltpu.VMEM((b_per_w,), jnp.int32),
            pltpu.VMEM((_CH, _HID), jnp.float32),
            pltpu.SemaphoreType.DMA,
        ],
    )
    def k(table_hbm, idx_hbm, out_hbm, idx_v, rows_v, sem):
        wid = lax.axis_index("s") * _NC + lax.axis_index("c")
        base = wid * b_per_w
        pltpu.sync_copy(idx_hbm.at[pl.ds(base, b_per_w)], idx_v)

        @pl.loop(0, b_per_w, step=_CH)
        def _(off):
            pltpu.async_copy(table_hbm.at[idx_v.at[pl.ds(off, _CH)]],
                             rows_v, sem).wait()
            pltpu.sync_copy(rows_v, out_hbm.at[pl.ds(base + off, _CH)])

    return k(table, idx)


def kernel(input_ids, word_emb, pos_emb, ln_w, ln_b):
    ids = input_ids.astype(jnp.int32)
    b, s = ids.shape
    ids_p = jnp.pad(ids, ((0, 0), (0, _SPAD - s)), constant_values=_PAD)
    n = b * _SPAD

    pos, a, e = pl.pallas_call(
        _stage1_body,
        out_shape=[
            jax.ShapeDtypeStruct((b, _SPAD), jnp.int32),
            jax.ShapeDtypeStruct((b, _SPAD), jnp.float32),
            jax.ShapeDtypeStruct((b, _SPAD), jnp.float32),
        ],
    )(ids_p)

    pg = _sc_gather(pos_emb, pos.reshape(n))

    out = pl.pallas_call(
        _main_body,
        grid=(n // _TBLK,),
        in_specs=[
            pl.BlockSpec((_TBLK, 1), lambda i: (i, 0)),
            pl.BlockSpec((_TBLK, 1), lambda i: (i, 0)),
            pl.BlockSpec((_TBLK, 1), lambda i: (i, 0)),
            pl.BlockSpec((_VOCAB, _HID), lambda i: (0, 0)),
            pl.BlockSpec((1, _HID), lambda i: (0, 0)),
            pl.BlockSpec((1, _HID), lambda i: (0, 0)),
            pl.BlockSpec((_TBLK, _HID), lambda i: (i, 0)),
        ],
        out_specs=pl.BlockSpec((_TBLK, _HID), lambda i: (i, 0)),
        out_shape=jax.ShapeDtypeStruct((n, _HID), jnp.float32),
    )(
        ids_p.reshape(n, 1),
        a.reshape(n, 1),
        e.reshape(n, 1),
        word_emb,
        ln_w.reshape(1, _HID),
        ln_b.reshape(1, _HID),
        pg,
    )
    return out.reshape(b, _SPAD, _HID)[:, :s, :]


# speedup vs baseline: 1.6159x; 1.0234x over previous
import functools

import jax
import jax.numpy as jnp
from jax import lax
from jax.experimental import pallas as pl
from jax.experimental.pallas import tpu as pltpu
from jax.experimental.pallas import tpu_sc as plsc

_VOCAB = 26
_HID = 640
_PAD = 1
_MASK = 24
_SPAD = 1024
_EPS = 1e-12
_MASK_RATIO_TRAIN = 0.15 * 0.8

_NC = 2
_NS = 16
_NW = _NC * _NS
_CH = 64

_TBLK = 512


def _stage1_body(ids_ref, pos_ref, a_ref, e_ref):
    ids = ids_ref[...]
    maskf = (ids != _PAD).astype(jnp.float32)
    s = ids.shape[1]
    row = lax.broadcasted_iota(jnp.int32, (s, s), 0)
    col = lax.broadcasted_iota(jnp.int32, (s, s), 1)
    tri = (row <= col).astype(jnp.bfloat16)
    inc = jnp.dot(maskf.astype(jnp.bfloat16), tri,
                  preferred_element_type=jnp.float32)
    pos_ref[...] = (inc * maskf + 2.0).astype(jnp.int32)
    ismask = ids == _MASK
    nmask = jnp.sum(ismask.astype(jnp.float32), axis=1, keepdims=True)
    srclen = jnp.sum(maskf, axis=1, keepdims=True)
    scale = (1.0 - _MASK_RATIO_TRAIN) / (1.0 - nmask / srclen)
    a_ref[...] = jnp.where(ismask, 0.0, scale)
    e_ref[...] = maskf


def _main_body(ids_ref, a_ref, e_ref, w_ref, lnw_ref, lnb_ref, pg_ref, out_ref):
    ids = ids_ref[...]
    voc = lax.broadcasted_iota(jnp.int32, (1, _VOCAB), 1)
    oh = (ids == voc).astype(jnp.float32)
    wg = jnp.dot(oh, w_ref[...], preferred_element_type=jnp.float32)
    x = a_ref[...] * wg + pg_ref[...]
    mu = jnp.mean(x, axis=1, keepdims=True)
    xc = x - mu
    var = jnp.mean(xc * xc, axis=1, keepdims=True)
    y = xc * lax.rsqrt(var + _EPS) * lnw_ref[...] + lnb_ref[...]
    out_ref[...] = (y * e_ref[...])[None]


def _sc_gather(table, idx):
    n = idx.shape[0]
    b_per_w = n // _NW
    mesh = plsc.VectorSubcoreMesh(core_axis_name="c", subcore_axis_name="s")

    nch = b_per_w // _CH

    @functools.partial(
        pl.kernel,
        mesh=mesh,
        out_type=jax.ShapeDtypeStruct((n, _HID), jnp.float32),
        scratch_types=[
            pltpu.VMEM((b_per_w,), jnp.int32),
            pltpu.VMEM((_CH, _HID), jnp.float32),
            pltpu.VMEM((_CH, _HID), jnp.float32),
            pltpu.SemaphoreType.DMA,
            pltpu.SemaphoreType.DMA,
            pltpu.SemaphoreType.DMA,
            pltpu.SemaphoreType.DMA,
        ],
    )
    def k(table_hbm, idx_hbm, out_hbm, idx_v, buf0, buf1, g0, g1, w0, w1):
        wid = lax.axis_index("s") * _NC + lax.axis_index("c")
        base = wid * b_per_w
        pltpu.sync_copy(idx_hbm.at[pl.ds(base, b_per_w)], idx_v)

        def start_g(ci, buf, sem):
            pltpu.async_copy(table_hbm.at[idx_v.at[pl.ds(ci * _CH, _CH)]],
                             buf, sem)

        def wait_g(ci, buf, sem):
            pltpu.make_async_copy(table_hbm.at[idx_v.at[pl.ds(ci * _CH, _CH)]],
                                  buf, sem).wait()

        def start_w(ci, buf, sem):
            pltpu.async_copy(buf, out_hbm.at[pl.ds(base + ci * _CH, _CH)], sem)

        def wait_w(ci, buf, sem):
            pltpu.make_async_copy(buf, out_hbm.at[pl.ds(base + ci * _CH, _CH)],
                                  sem).wait()

        start_g(0, buf0, g0)
        start_g(1, buf1, g1)

        @pl.loop(0, nch, step=2)
        def _(ci):
            wait_g(ci, buf0, g0)
            start_w(ci, buf0, w0)
            wait_g(ci + 1, buf1, g1)
            wait_w(ci, buf0, w0)

            @pl.when(ci + 2 < nch)
            def _():
                start_g(ci + 2, buf0, g0)

            start_w(ci + 1, buf1, w1)
            wait_w(ci + 1, buf1, w1)

            @pl.when(ci + 3 < nch)
            def _():
                start_g(ci + 3, buf1, g1)

    return k(table, idx)


def kernel(input_ids, word_emb, pos_emb, ln_w, ln_b):
    ids = input_ids.astype(jnp.int32)
    b, s = ids.shape
    ids_p = jnp.pad(ids, ((0, 0), (0, _SPAD - s)), constant_values=_PAD)
    n = b * _SPAD

    pos, a, e = pl.pallas_call(
        _stage1_body,
        out_shape=[
            jax.ShapeDtypeStruct((b, _SPAD), jnp.int32),
            jax.ShapeDtypeStruct((b, _SPAD), jnp.float32),
            jax.ShapeDtypeStruct((b, _SPAD), jnp.float32),
        ],
    )(ids_p)

    pg = _sc_gather(pos_emb, pos.reshape(n))

    nj = _SPAD // _TBLK
    out = pl.pallas_call(
        _main_body,
        grid=(b, nj),
        in_specs=[
            pl.BlockSpec((_TBLK, 1), lambda i, j: (i * nj + j, 0)),
            pl.BlockSpec((_TBLK, 1), lambda i, j: (i * nj + j, 0)),
            pl.BlockSpec((_TBLK, 1), lambda i, j: (i * nj + j, 0)),
            pl.BlockSpec((_VOCAB, _HID), lambda i, j: (0, 0)),
            pl.BlockSpec((1, _HID), lambda i, j: (0, 0)),
            pl.BlockSpec((1, _HID), lambda i, j: (0, 0)),
            pl.BlockSpec((_TBLK, _HID), lambda i, j: (i * nj + j, 0)),
        ],
        out_specs=pl.BlockSpec((1, _TBLK, _HID), lambda i, j: (i, j, 0)),
        out_shape=jax.ShapeDtypeStruct((b, s, _HID), jnp.float32),
    )(
        ids_p.reshape(n, 1),
        a.reshape(n, 1),
        e.reshape(n, 1),
        word_emb,
        ln_w.reshape(1, _HID),
        ln_b.reshape(1, _HID),
        pg,
    )
    return out
